# Initial kernel scaffold; baseline (speedup 1.0000x reference)
import jax
import jax.numpy as jnp
from jax.experimental import pallas as pl


def kernel(predict, target):
    raise NotImplementedError("write your pallas kernel here")


# speedup vs baseline: 2.1950x; 2.1950x over previous
import functools

import jax
import jax.numpy as jnp
from jax import lax
from jax.experimental import pallas as pl
from jax.experimental.pallas import tpu as pltpu

_THEA = 0.5
_DELTA = 1.5
_NCLS = 19
_CP = 32


def _body(pred_ref, tgt_ref, out_ref, sums, counts, stats):
    p = pl.program_id(0)
    i = pl.program_id(1)
    b = pl.program_id(2)
    ilast = pl.num_programs(1) - 1
    blast = pl.num_programs(2) - 1

    @pl.when((p == 0) & (i == 0) & (b == 0))
    def _init():
        sums[...] = jnp.zeros_like(sums)
        counts[...] = jnp.zeros_like(counts)
        stats[...] = jnp.zeros_like(stats)

    pred = pred_ref[0]
    tgt = tgt_ref[0]
    pp = pred.shape[1]
    cls = lax.broadcasted_iota(jnp.int32, (_CP, pp), 0)
    oh = (cls == tgt).astype(jnp.float32)

    @pl.when(p == 0)
    def _pass1():
        sums[...] += lax.dot_general(
            oh, pred, (((1,), (1,)), ((), ())),
            preferred_element_type=jnp.float32,
            precision=lax.Precision.HIGHEST)
        counts[...] += jnp.sum(oh, axis=1, keepdims=True)

    @pl.when(p == 1)
    def _pass2():
        cnt_safe = jnp.maximum(counts[...], 1.0)
        centers = sums[...] / cnt_safe
        cpx = lax.dot_general(
            centers, oh, (((0,), (0,)), ((), ())),
            preferred_element_type=jnp.float32,
            precision=lax.Precision.HIGHEST)
        diff = pred - cpx
        d2 = jnp.sum(diff * diff, axis=0, keepdims=True)
        d = jnp.sqrt(d2)
        r = jnp.maximum(d - _THEA, 0.0)
        rsq = r * r
        act = (r > 0.0).astype(jnp.float32)
        stacked = jnp.concatenate([rsq, act], axis=0)
        stats[...] += lax.dot_general(
            oh, stacked, (((1,), (1,)), ((), ())),
            preferred_element_type=jnp.float32,
            precision=lax.Precision.HIGHEST)

    @pl.when((p == 1) & (i == ilast) & (b == blast))
    def _final():
        cnt = counts[...]
        validf = (cnt > 20.0).astype(jnp.float32)
        nv = jnp.sum(validf)
        nv_safe = jnp.maximum(nv, 1.0)
        sum_sq = stats[:, 0:1]
        pos = jnp.maximum(stats[:, 1:2], 1.0)
        loss_var = jnp.sum(validf * sum_sq / pos) / nv_safe

        centers = sums[...] / jnp.maximum(cnt, 1.0)
        diff3 = centers[:, None, :] - centers[None, :, :]
        d2m = jnp.sum(diff3 * diff3, axis=2)
        dm = jnp.sqrt(d2m)
        rm = jnp.maximum(2.0 * _DELTA - dm, 0.0)
        pairmask = lax.dot_general(
            validf, validf, (((1,), (1,)), ((), ())),
            preferred_element_type=jnp.float32,
            precision=lax.Precision.HIGHEST)
        row = lax.broadcasted_iota(jnp.int32, (_CP, _CP), 0)
        col = lax.broadcasted_iota(jnp.int32, (_CP, _CP), 1)
        offdiag = (row != col).astype(jnp.float32)
        loss_dis = jnp.sum(rm * rm * pairmask * offdiag)
        loss_dis = loss_dis / jnp.maximum(nv * (nv - 1.0), 1.0)

        cnorm = jnp.sqrt(jnp.sum(centers * centers, axis=1, keepdims=True))
        loss_reg = jnp.sum(validf * cnorm) / nv_safe

        total = loss_var + loss_dis + 0.001 * loss_reg
        out_ref[...] = total.reshape(1, 1)


def kernel(predict, target):
    n, c, h, w = predict.shape
    npix = h * w
    pblk = 8192
    nblk = npix // pblk
    pred = predict.reshape(n, c, npix)
    tgt = target.reshape(n * nblk, 1, pblk)
    out = pl.pallas_call(
        _body,
        grid=(2, n, nblk),
        in_specs=[
            pl.BlockSpec((1, c, pblk), lambda p, i, b: (i, 0, b)),
            pl.BlockSpec((1, 1, pblk),
                         lambda p, i, b, nb=nblk: (i * nb + b, 0, 0)),
        ],
        out_specs=pl.BlockSpec((1, 1), lambda p, i, b: (0, 0)),
        out_shape=jax.ShapeDtypeStruct((1, 1), jnp.float32),
        scratch_shapes=[
            pltpu.VMEM((_CP, c), jnp.float32),
            pltpu.VMEM((_CP, 1), jnp.float32),
            pltpu.VMEM((_CP, 2), jnp.float32),
        ],
    )(pred, tgt)
    return out[0, 0]


# speedup vs baseline: 3.8841x; 1.7695x over previous
import jax
import jax.numpy as jnp
from jax import lax
from jax.experimental import pallas as pl
from jax.experimental.pallas import tpu as pltpu

_THEA = 0.5
_DELTA = 1.5
_NCLS = 19
_CP = 32


def _dot(a, b, dims):
    return lax.dot_general(a, b, (dims, ((), ())),
                           preferred_element_type=jnp.float32)


def _body(pred_ref, tgt_ref, out_ref, sums, stats):
    p = pl.program_id(0)
    i = pl.program_id(1)
    b = pl.program_id(2)
    ilast = pl.num_programs(1) - 1
    blast = pl.num_programs(2) - 1

    @pl.when((p == 0) & (i == 0) & (b == 0))
    def _init():
        sums[...] = jnp.zeros_like(sums)
        stats[...] = jnp.zeros_like(stats)

    pred = pred_ref[0]
    tgt = tgt_ref[0]
    pp = pred.shape[1]
    cls = lax.broadcasted_iota(jnp.int32, (_CP, pp), 0)
    oh = (cls == tgt).astype(jnp.bfloat16)

    @pl.when(p == 0)
    def _pass1():
        pred_aug = jnp.concatenate(
            [pred.astype(jnp.bfloat16),
             jnp.ones((1, pp), jnp.bfloat16)], axis=0)
        sums[...] += _dot(oh, pred_aug, (((1,), (1,))))

    @pl.when(p == 1)
    def _pass2():
        cnt = sums[:, -1:]
        centers = sums[:, :-1] / jnp.maximum(cnt, 1.0)
        cpx = _dot(centers.astype(jnp.bfloat16), oh,
                   (((0,), (0,))))
        diff = pred - cpx
        d2 = jnp.sum(diff * diff, axis=0, keepdims=True)
        d = jnp.sqrt(d2)
        r = jnp.maximum(d - _THEA, 0.0)
        rsq = (r * r).astype(jnp.bfloat16)
        act = (r > 0.0).astype(jnp.bfloat16)
        stacked = jnp.concatenate([rsq, act], axis=0)
        stats[...] += _dot(oh, stacked, (((1,), (1,))))

    @pl.when((p == 1) & (i == ilast) & (b == blast))
    def _final():
        cnt = sums[:, -1:]
        validf = (cnt > 20.0).astype(jnp.float32)
        nv = jnp.sum(validf)
        nv_safe = jnp.maximum(nv, 1.0)
        sum_sq = stats[:, 0:1]
        pos = jnp.maximum(stats[:, 1:2], 1.0)
        loss_var = jnp.sum(validf * sum_sq / pos) / nv_safe

        centers = sums[:, :-1] / jnp.maximum(cnt, 1.0)
        diff3 = centers[:, None, :] - centers[None, :, :]
        d2m = jnp.sum(diff3 * diff3, axis=2)
        dm = jnp.sqrt(d2m)
        rm = jnp.maximum(2.0 * _DELTA - dm, 0.0)
        pairmask = lax.dot_general(
            validf, validf, (((1,), (1,)), ((), ())),
            preferred_element_type=jnp.float32)
        row = lax.broadcasted_iota(jnp.int32, (_CP, _CP), 0)
        col = lax.broadcasted_iota(jnp.int32, (_CP, _CP), 1)
        offdiag = (row != col).astype(jnp.float32)
        loss_dis = jnp.sum(rm * rm * pairmask * offdiag)
        loss_dis = loss_dis / jnp.maximum(nv * (nv - 1.0), 1.0)

        cnorm = jnp.sqrt(jnp.sum(centers * centers, axis=1, keepdims=True))
        loss_reg = jnp.sum(validf * cnorm) / nv_safe

        total = loss_var + loss_dis + 0.001 * loss_reg
        out_ref[...] = total.reshape(1, 1)


def kernel(predict, target):
    n, c, h, w = predict.shape
    npix = h * w
    pblk = 16384
    nblk = npix // pblk
    pred = predict.reshape(n, c, npix)
    tgt = target.reshape(n * nblk, 1, pblk)
    out = pl.pallas_call(
        _body,
        grid=(2, n, nblk),
        in_specs=[
            pl.BlockSpec((1, c, pblk), lambda p, i, b: (i, 0, b)),
            pl.BlockSpec((1, 1, pblk),
                         lambda p, i, b, nb=nblk: (i * nb + b, 0, 0)),
        ],
        out_specs=pl.BlockSpec((1, 1), lambda p, i, b: (0, 0)),
        out_shape=jax.ShapeDtypeStruct((1, 1), jnp.float32),
        scratch_shapes=[
            pltpu.VMEM((_CP, c + 1), jnp.float32),
            pltpu.VMEM((_CP, 2), jnp.float32),
        ],
    )(pred, tgt)
    return out[0, 0]
